# Initial kernel scaffold; baseline (speedup 1.0000x reference)
import jax
import jax.numpy as jnp
from jax.experimental import pallas as pl


def kernel(heads, relations, tails, entity_emb, relation_emb):
    raise NotImplementedError("write your pallas kernel here")


# speedup vs baseline: 1.1294x; 1.1294x over previous
import functools

import jax
import jax.numpy as jnp
from jax import lax
from jax.experimental import pallas as pl
from jax.experimental.pallas import tpu as pltpu
from jax.experimental.pallas import tpu_sc as plsc

DIM = 64
LANES = 16
CHUNK = 128
BLOCKS = CHUNK // LANES


def _sqrt(s):
    bits = lax.bitcast_convert_type(s, jnp.int32)
    y = lax.bitcast_convert_type(
        jnp.int32(0x5F3759DF) - lax.shift_right_logical(bits, 1), jnp.float32)
    for _ in range(3):
        y = y * (1.5 - 0.5 * s * y * y)
    return s * y


def _sc_kernel(batch, n_workers):
    rows_per_worker = batch // n_workers
    n_chunks = rows_per_worker // CHUNK
    mesh = plsc.VectorSubcoreMesh(core_axis_name="c", subcore_axis_name="s")

    @functools.partial(
        pl.kernel,
        mesh=mesh,
        compiler_params=pltpu.CompilerParams(
            needs_layout_passes=False, use_tc_tiling_on_sc=False),
        out_type=jax.ShapeDtypeStruct((batch,), jnp.float32),
        scratch_types=[
            pltpu.VMEM((CHUNK,), jnp.int32),
            pltpu.VMEM((CHUNK,), jnp.int32),
            pltpu.VMEM((CHUNK,), jnp.int32),
            pltpu.VMEM((CHUNK, DIM), jnp.float32),
            pltpu.VMEM((CHUNK, DIM), jnp.float32),
            pltpu.VMEM((CHUNK, DIM), jnp.float32),
            pltpu.VMEM((LANES * LANES,), jnp.float32),
            pltpu.VMEM((CHUNK,), jnp.float32),
            pltpu.SemaphoreType.DMA,
        ],
    )
    def k(heads, relations, tails, entity, relation, out,
          hidx, ridx, tidx, hrows, rrows, trows, st, outc, sem):
        n_cores = 2
        wid = lax.axis_index("s") * n_cores + lax.axis_index("c")
        base = wid * rows_per_worker
        lane_iota = lax.iota(jnp.int32, LANES)

        def chunk_body(c, _):
            off = base + c * CHUNK
            pltpu.sync_copy(heads.at[pl.ds(off, CHUNK)], hidx)
            pltpu.sync_copy(relations.at[pl.ds(off, CHUNK)], ridx)
            pltpu.sync_copy(tails.at[pl.ds(off, CHUNK)], tidx)
            ch = pltpu.async_copy(entity.at[hidx], hrows, sem)
            cr = pltpu.async_copy(relation.at[ridx], rrows, sem)
            ct = pltpu.async_copy(entity.at[tidx], trows, sem)
            ch.wait()
            cr.wait()
            ct.wait()

            def blk_body(b, _):
                r0 = b * LANES
                for u in range(LANES):
                    r = r0 + u
                    s = jnp.zeros((LANES,), jnp.float32)
                    for kk in range(DIM // LANES):
                        sl = pl.ds(kk * LANES, LANES)
                        d = hrows[r, sl] + rrows[r, sl] - trows[r, sl]
                        s = s + d * d
                    plsc.store_scatter(st, [lane_iota * LANES + u], s)
                acc = st[pl.ds(0, LANES)]
                for j in range(1, LANES):
                    acc = acc + st[pl.ds(j * LANES, LANES)]
                outc[pl.ds(r0, LANES)] = _sqrt(acc)
                return 0

            lax.fori_loop(0, BLOCKS, blk_body, 0)
            pltpu.sync_copy(outc, out.at[pl.ds(off, CHUNK)])
            return 0

        lax.fori_loop(0, n_chunks, chunk_body, 0)

    return k


def kernel(heads, relations, tails, entity_emb, relation_emb):
    batch = heads.shape[0]
    k = _sc_kernel(batch, 32)
    return k(heads.astype(jnp.int32), relations.astype(jnp.int32),
             tails.astype(jnp.int32), entity_emb, relation_emb)
